# baseline (speedup 1.0000x reference)
import functools

import jax
import jax.numpy as jnp
from jax import lax
from jax.experimental import pallas as pl
from jax.experimental.pallas import tpu as pltpu, tpu_sc as plsc

C = 2600
B = 4096
L = 16
NC = 2
NS = 16
NW = NC * NS
BPW = B // NW
CB = 128
NFULL = C // CB
TAIL = C - NFULL * CB
BT = 512
NBT = B // BT

_STEPS = [2048, 1024, 512, 256, 128, 64, 32, 16, 8, 4, 2, 1]


def _sc_body(y_hbm, uniq_hbm, anc_hbm, dec_hbm, mod_hbm,
             ml_hbm, dnc_hbm, mnc_hbm, pred_hbm,
             uniq_v, idx_v, bufs, sems):
    wid = lax.axis_index("s") * NC + lax.axis_index("c")
    base = wid * BPW

    pltpu.sync_copy(y_hbm.at[pl.ds(base, BPW)], idx_v)
    pltpu.sync_copy(uniq_hbm, uniq_v)

    for i in range(BPW // L):
        y = idx_v[pl.ds(i * L, L)]
        pos = jnp.zeros((L,), jnp.int32)
        for s in _STEPS:
            cand = pos + s
            ok = cand <= C
            gidx = jnp.where(ok, cand, 1) - 1
            val = plsc.load_gather(uniq_v, [gidx])
            pos = jnp.where(ok & (val < y), cand, pos)
        idx_v[pl.ds(i * L, L)] = pos

    pltpu.sync_copy(idx_v, pred_hbm.at[pl.ds(base, BPW)])

    tbls = (anc_hbm, dec_hbm, mod_hbm)
    outs = (ml_hbm, dnc_hbm, mnc_hbm)

    def start_gather(t, b, col):
        pltpu.async_copy(tbls[t].at[:, pl.ds(col, CB)].at[idx_v],
                         bufs.at[t * 2 + b], sems.at[t * 2 + b])

    def wait_gather(t, b):
        pltpu.make_async_copy(
            tbls[t].at[:, pl.ds(0, CB)].at[idx_v],
            bufs.at[t * 2 + b], sems.at[t * 2 + b]).wait()

    def start_scatter(t, b, col):
        pltpu.async_copy(bufs.at[t * 2 + b],
                         outs[t].at[pl.ds(base, BPW), pl.ds(col, CB)],
                         sems.at[6 + t * 2 + b])

    def wait_scatter(t, b):
        pltpu.make_async_copy(
            bufs.at[t * 2 + b],
            outs[t].at[pl.ds(base, BPW), pl.ds(0, CB)],
            sems.at[6 + t * 2 + b]).wait()

    for t in range(3):
        start_gather(t, 0, 0)

    def pair_body(j, carry):
        for b in range(2):
            i = 2 * j + b
            col = i * CB
            for t in range(3):
                wait_gather(t, b)
                start_scatter(t, b, col)
            for t in range(3):
                if b == 0:
                    @pl.when(j >= 1)
                    def _():
                        wait_scatter(t, 1)
                    start_gather(t, 1, col + CB)
                else:
                    wait_scatter(t, 0)

                    @pl.when(j < NFULL // 2 - 1)
                    def _():
                        start_gather(t, 0, col + CB)
        return carry

    lax.fori_loop(0, NFULL // 2, pair_body, 0)

    for t in range(3):
        wait_scatter(t, 1)


def _tc_tail_body(idx_ref, ta_ref, td_ref, tm_ref,
                  mli_ref, dnci_ref, mnci_ref,
                  mlo_ref, dnco_ref, mnco_ref):
    del mli_ref, dnci_ref, mnci_ref
    idx = idx_ref[0, 0, :]
    onehot = (lax.broadcasted_iota(jnp.int32, (BT, C), 1)
              == idx[:, None]).astype(jnp.float32)
    mlo_ref[...] = jnp.dot(onehot, ta_ref[...],
                           preferred_element_type=jnp.float32)
    dnco_ref[...] = jnp.dot(onehot, td_ref[...],
                            preferred_element_type=jnp.float32)
    mnco_ref[...] = jnp.dot(onehot, tm_ref[...],
                            preferred_element_type=jnp.float32)


def kernel(y_n, unique_cell_types, ancestors, descendents, mod):
    out_type = (
        jax.ShapeDtypeStruct((B, C), jnp.float32),
        jax.ShapeDtypeStruct((B, C), jnp.float32),
        jax.ShapeDtypeStruct((B, C), jnp.float32),
        jax.ShapeDtypeStruct((B,), jnp.int32),
    )
    sc = pl.kernel(
        _sc_body,
        out_type=out_type,
        mesh=plsc.VectorSubcoreMesh(core_axis_name="c", subcore_axis_name="s"),
        compiler_params=pltpu.CompilerParams(needs_layout_passes=False,
                                             use_tc_tiling_on_sc=True),
        scratch_types=[
            pltpu.VMEM((C,), jnp.int32),
            pltpu.VMEM((BPW,), jnp.int32),
            pltpu.VMEM((6, BPW, CB), jnp.float32),
            pltpu.SemaphoreType.DMA((12,)),
        ],
    )
    ml, dnc, mnc, pred = sc(y_n, unique_cell_types, ancestors, descendents,
                            mod)

    idx3 = pred.reshape(NBT, 1, BT)
    tail_spec = pl.BlockSpec((C, CB), lambda i: (0, NFULL))
    big_spec = pl.BlockSpec(memory_space=pl.ANY)
    out_spec = pl.BlockSpec((BT, CB), lambda i: (i, NFULL))
    tc = pl.pallas_call(
        _tc_tail_body,
        grid=(NBT,),
        in_specs=[
            pl.BlockSpec((1, 1, BT), lambda i: (i, 0, 0)),
            tail_spec, tail_spec, tail_spec,
            big_spec, big_spec, big_spec,
        ],
        out_specs=(out_spec, out_spec, out_spec),
        out_shape=(
            jax.ShapeDtypeStruct((B, C), jnp.float32),
            jax.ShapeDtypeStruct((B, C), jnp.float32),
            jax.ShapeDtypeStruct((B, C), jnp.float32),
        ),
        input_output_aliases={4: 0, 5: 1, 6: 2},
    )
    ml, dnc, mnc = tc(idx3, ancestors, descendents, mod, ml, dnc, mnc)
    return (ml, dnc, mnc, pred)


# speedup vs baseline: 1.0115x; 1.0115x over previous
import functools

import jax
import jax.numpy as jnp
from jax import lax
from jax.experimental import pallas as pl
from jax.experimental.pallas import tpu as pltpu, tpu_sc as plsc

C = 2600
B = 4096
L = 16
NC = 2
NS = 16
NW = NC * NS
BPW = B // NW
CB = 128
NFULL = C // CB
TAIL = C - NFULL * CB
BT = 512
NBT = B // BT

_STEPS = [2048, 1024, 512, 256, 128, 64, 32, 16, 8, 4, 2, 1]


def _sc_body(y_hbm, uniq_hbm, anc_hbm, dec_hbm, mod_hbm,
             ml_hbm, dnc_hbm, mnc_hbm, pred_hbm,
             uniq_v, idx_v, bufs, sems):
    wid = lax.axis_index("s") * NC + lax.axis_index("c")
    base = wid * BPW

    pltpu.sync_copy(y_hbm.at[pl.ds(base, BPW)], idx_v)
    pltpu.sync_copy(uniq_hbm, uniq_v)

    for i in range(BPW // L):
        y = idx_v[pl.ds(i * L, L)]
        pos = jnp.zeros((L,), jnp.int32)
        for s in _STEPS:
            cand = pos + s
            ok = cand <= C
            gidx = jnp.where(ok, cand, 1) - 1
            val = plsc.load_gather(uniq_v, [gidx])
            pos = jnp.where(ok & (val < y), cand, pos)
        idx_v[pl.ds(i * L, L)] = pos

    pltpu.sync_copy(idx_v, pred_hbm.at[pl.ds(base, BPW)])

    tbls = (anc_hbm, dec_hbm, mod_hbm)
    outs = (ml_hbm, dnc_hbm, mnc_hbm)
    HR = BPW // 2
    NT = 2 * NFULL

    def start_gather(t, s, col, h):
        pltpu.async_copy(
            tbls[t].at[:, pl.ds(col, CB)].at[idx_v.at[pl.ds(h * HR, HR)]],
            bufs.at[t * 4 + s], sems.at[t * 4 + s])

    def wait_gather(t, s):
        pltpu.make_async_copy(
            tbls[t].at[:, pl.ds(0, CB)].at[idx_v.at[pl.ds(0, HR)]],
            bufs.at[t * 4 + s], sems.at[t * 4 + s]).wait()

    def start_scatter(t, s, col, h):
        pltpu.async_copy(bufs.at[t * 4 + s],
                         outs[t].at[pl.ds(base + h * HR, HR), pl.ds(col, CB)],
                         sems.at[12 + t * 4 + s])

    def wait_scatter(t, s):
        pltpu.make_async_copy(
            bufs.at[t * 4 + s],
            outs[t].at[pl.ds(base, HR), pl.ds(0, CB)],
            sems.at[12 + t * 4 + s]).wait()

    for t in range(3):
        start_gather(t, 0, 0, 0)
        start_gather(t, 1, 0, 1)

    def quad_body(j, carry):
        for u in range(4):
            col = (2 * j + u // 2) * CB
            h = u % 2
            for t in range(3):
                wait_gather(t, u)
                start_scatter(t, u, col, h)
            for t in range(3):
                if u < 2:
                    @pl.when(j >= 1)
                    def _():
                        wait_scatter(t, u + 2)
                    start_gather(t, u + 2, col + CB, h)
                else:
                    wait_scatter(t, u - 2)

                    @pl.when(j < NT // 4 - 1)
                    def _():
                        start_gather(t, u - 2, col + CB, h)
        return carry

    lax.fori_loop(0, NT // 4, quad_body, 0)

    for t in range(3):
        wait_scatter(t, 2)
        wait_scatter(t, 3)


def _tc_tail_body(idx_ref, ta_ref, td_ref, tm_ref,
                  mli_ref, dnci_ref, mnci_ref,
                  mlo_ref, dnco_ref, mnco_ref):
    del mli_ref, dnci_ref, mnci_ref
    idx = idx_ref[0, 0, :]
    onehot = (lax.broadcasted_iota(jnp.int32, (BT, C), 1)
              == idx[:, None]).astype(jnp.float32)
    mlo_ref[...] = jnp.dot(onehot, ta_ref[...],
                           preferred_element_type=jnp.float32)
    dnco_ref[...] = jnp.dot(onehot, td_ref[...],
                            preferred_element_type=jnp.float32)
    mnco_ref[...] = jnp.dot(onehot, tm_ref[...],
                            preferred_element_type=jnp.float32)


def kernel(y_n, unique_cell_types, ancestors, descendents, mod):
    out_type = (
        jax.ShapeDtypeStruct((B, C), jnp.float32),
        jax.ShapeDtypeStruct((B, C), jnp.float32),
        jax.ShapeDtypeStruct((B, C), jnp.float32),
        jax.ShapeDtypeStruct((B,), jnp.int32),
    )
    sc = pl.kernel(
        _sc_body,
        out_type=out_type,
        mesh=plsc.VectorSubcoreMesh(core_axis_name="c", subcore_axis_name="s"),
        compiler_params=pltpu.CompilerParams(needs_layout_passes=False,
                                             use_tc_tiling_on_sc=True),
        scratch_types=[
            pltpu.VMEM((C,), jnp.int32),
            pltpu.VMEM((BPW,), jnp.int32),
            pltpu.VMEM((12, BPW // 2, CB), jnp.float32),
            pltpu.SemaphoreType.DMA((24,)),
        ],
    )
    ml, dnc, mnc, pred = sc(y_n, unique_cell_types, ancestors, descendents,
                            mod)

    idx3 = pred.reshape(NBT, 1, BT)
    tail_spec = pl.BlockSpec((C, CB), lambda i: (0, NFULL))
    big_spec = pl.BlockSpec(memory_space=pl.ANY)
    out_spec = pl.BlockSpec((BT, CB), lambda i: (i, NFULL))
    tc = pl.pallas_call(
        _tc_tail_body,
        grid=(NBT,),
        in_specs=[
            pl.BlockSpec((1, 1, BT), lambda i: (i, 0, 0)),
            tail_spec, tail_spec, tail_spec,
            big_spec, big_spec, big_spec,
        ],
        out_specs=(out_spec, out_spec, out_spec),
        out_shape=(
            jax.ShapeDtypeStruct((B, C), jnp.float32),
            jax.ShapeDtypeStruct((B, C), jnp.float32),
            jax.ShapeDtypeStruct((B, C), jnp.float32),
        ),
        input_output_aliases={4: 0, 5: 1, 6: 2},
    )
    ml, dnc, mnc = tc(idx3, ancestors, descendents, mod, ml, dnc, mnc)
    return (ml, dnc, mnc, pred)


# speedup vs baseline: 1.0818x; 1.0694x over previous
import functools

import jax
import jax.numpy as jnp
from jax import lax
from jax.experimental import pallas as pl
from jax.experimental.pallas import tpu as pltpu, tpu_sc as plsc

C = 2600
B = 4096
L = 16
NC = 2
NS = 16
NW = NC * NS
BPW = B // NW
CB = 128
NFULL = C // CB
TAIL = C - NFULL * CB
BT = 512
NBT = B // BT

_STEPS = [2048, 1024, 512, 256, 128, 64, 32, 16, 8, 4, 2, 1]


def _sc_body(y_hbm, uniq_hbm, anc_hbm, dec_hbm, mod_hbm,
             ml_hbm, dnc_hbm, mnc_hbm, pred_hbm,
             uniq_v, idx_v, bufs, sems):
    wid = lax.axis_index("s") * NC + lax.axis_index("c")
    base = wid * BPW

    pltpu.sync_copy(y_hbm.at[pl.ds(base, BPW)], idx_v)
    pltpu.sync_copy(uniq_hbm, uniq_v)

    for i in range(BPW // L):
        y = idx_v[pl.ds(i * L, L)]
        pos = jnp.zeros((L,), jnp.int32)
        for s in _STEPS:
            cand = pos + s
            ok = cand <= C
            gidx = jnp.where(ok, cand, 1) - 1
            val = plsc.load_gather(uniq_v, [gidx])
            pos = jnp.where(ok & (val < y), cand, pos)
        idx_v[pl.ds(i * L, L)] = pos

    pltpu.sync_copy(idx_v, pred_hbm.at[pl.ds(base, BPW)])

    tbls = (anc_hbm, dec_hbm, mod_hbm)
    outs = (ml_hbm, dnc_hbm, mnc_hbm)
    HR = BPW // 2
    NT = 2 * NFULL

    def start_gather(t, s, col, h):
        pltpu.async_copy(
            tbls[t].at[:, pl.ds(col, CB)].at[idx_v.at[pl.ds(h * HR, HR)]],
            bufs.at[t * 4 + s], sems.at[t * 4 + s])

    def wait_gather(t, s):
        pltpu.make_async_copy(
            tbls[t].at[:, pl.ds(0, CB)].at[idx_v.at[pl.ds(0, HR)]],
            bufs.at[t * 4 + s], sems.at[t * 4 + s]).wait()

    def start_scatter(t, s, col, h):
        pltpu.async_copy(bufs.at[t * 4 + s],
                         outs[t].at[pl.ds(base + h * HR, HR), pl.ds(col, CB)],
                         sems.at[12 + t * 4 + s])

    def wait_scatter(t, s):
        pltpu.make_async_copy(
            bufs.at[t * 4 + s],
            outs[t].at[pl.ds(base, HR), pl.ds(0, CB)],
            sems.at[12 + t * 4 + s]).wait()

    for t in range(3):
        start_gather(t, 0, 0, 0)
        start_gather(t, 1, 0, 1)

    def quad_body(j, carry):
        for u in range(4):
            col = (2 * j + u // 2) * CB
            h = u % 2
            for t in range(3):
                wait_gather(t, u)
                start_scatter(t, u, col, h)
            for t in range(3):
                if u < 2:
                    @pl.when(j >= 1)
                    def _():
                        wait_scatter(t, u + 2)
                    start_gather(t, u + 2, col + CB, h)
                else:
                    wait_scatter(t, u - 2)

                    @pl.when(j < NT // 4 - 1)
                    def _():
                        start_gather(t, u - 2, col + CB, h)
        return carry

    lax.fori_loop(0, NT // 4, quad_body, 0)

    for t in range(3):
        wait_scatter(t, 2)
        wait_scatter(t, 3)


def _tc_tail_body(idx_ref, ta_ref, td_ref, tm_ref,
                  mli_ref, dnci_ref, mnci_ref,
                  mlo_ref, dnco_ref, mnco_ref):
    del mli_ref, dnci_ref, mnci_ref
    idx = idx_ref[0, 0, :]
    onehot = (lax.broadcasted_iota(jnp.int32, (BT, C), 1)
              == idx[:, None]).astype(jnp.float32)
    mlo_ref[...] = jnp.dot(onehot, ta_ref[...],
                           preferred_element_type=jnp.float32)
    dnco_ref[...] = jnp.dot(onehot, td_ref[...],
                            preferred_element_type=jnp.float32)
    mnco_ref[...] = jnp.dot(onehot, tm_ref[...],
                            preferred_element_type=jnp.float32)


BI = 512
BJ = 512


def _tc_tr_body(mli_ref, dnci_ref, mnci_ref, mlo_ref, dnco_ref, mnco_ref):
    mlo_ref[...] = mli_ref[...].T
    dnco_ref[...] = dnci_ref[...].T
    mnco_ref[...] = mnci_ref[...].T


def kernel(y_n, unique_cell_types, ancestors, descendents, mod):
    out_type = (
        jax.ShapeDtypeStruct((B, C), jnp.float32),
        jax.ShapeDtypeStruct((B, C), jnp.float32),
        jax.ShapeDtypeStruct((B, C), jnp.float32),
        jax.ShapeDtypeStruct((B,), jnp.int32),
    )
    sc = pl.kernel(
        _sc_body,
        out_type=out_type,
        mesh=plsc.VectorSubcoreMesh(core_axis_name="c", subcore_axis_name="s"),
        compiler_params=pltpu.CompilerParams(needs_layout_passes=False,
                                             use_tc_tiling_on_sc=True),
        scratch_types=[
            pltpu.VMEM((C,), jnp.int32),
            pltpu.VMEM((BPW,), jnp.int32),
            pltpu.VMEM((12, BPW // 2, CB), jnp.float32),
            pltpu.SemaphoreType.DMA((24,)),
        ],
    )
    ml, dnc, mnc, pred = sc(y_n, unique_cell_types, ancestors, descendents,
                            mod)

    idx3 = pred.reshape(NBT, 1, BT)
    tail_spec = pl.BlockSpec((C, CB), lambda i: (0, NFULL))
    big_spec = pl.BlockSpec(memory_space=pl.ANY)
    out_spec = pl.BlockSpec((BT, CB), lambda i: (i, NFULL))
    tc = pl.pallas_call(
        _tc_tail_body,
        grid=(NBT,),
        in_specs=[
            pl.BlockSpec((1, 1, BT), lambda i: (i, 0, 0)),
            tail_spec, tail_spec, tail_spec,
            big_spec, big_spec, big_spec,
        ],
        out_specs=(out_spec, out_spec, out_spec),
        out_shape=(
            jax.ShapeDtypeStruct((B, C), jnp.float32),
            jax.ShapeDtypeStruct((B, C), jnp.float32),
            jax.ShapeDtypeStruct((B, C), jnp.float32),
        ),
        input_output_aliases={4: 0, 5: 1, 6: 2},
    )
    ml, dnc, mnc = tc(idx3, ancestors, descendents, mod, ml, dnc, mnc)

    in_spec = pl.BlockSpec((BI, BJ), lambda i, j: (i, j))
    tr_out_spec = pl.BlockSpec((BJ, BI), lambda i, j: (j, i))
    tr = pl.pallas_call(
        _tc_tr_body,
        grid=(B // BI, -(-C // BJ)),
        in_specs=[in_spec, in_spec, in_spec],
        out_specs=(tr_out_spec, tr_out_spec, tr_out_spec),
        out_shape=(
            jax.ShapeDtypeStruct((C, B), jnp.float32),
            jax.ShapeDtypeStruct((C, B), jnp.float32),
            jax.ShapeDtypeStruct((C, B), jnp.float32),
        ),
    )
    mlT, dncT, mncT = tr(ml, dnc, mnc)
    return (mlT.T, dncT.T, mncT.T, pred)


# speedup vs baseline: 1.1441x; 1.0576x over previous
import functools

import jax
import jax.numpy as jnp
from jax import lax
from jax.experimental import pallas as pl
from jax.experimental.pallas import tpu as pltpu, tpu_sc as plsc

C = 2600
B = 4096
L = 16
NC = 2
NS = 16
NW = NC * NS
BPW = B // NW
CB = 128
NFULL = C // CB
TAIL = C - NFULL * CB
BT = 512
NBT = B // BT

_STEPS = [2048, 1024, 512, 256, 128, 64, 32, 16, 8, 4, 2, 1]


def _sc_body(y_hbm, uniq_hbm, anc_hbm, dec_hbm, mod_hbm,
             ml_hbm, dnc_hbm, mnc_hbm, pred_hbm,
             uniq_v, idx_v, bufs, sems):
    wid = lax.axis_index("s") * NC + lax.axis_index("c")
    base = wid * BPW

    pltpu.sync_copy(y_hbm.at[pl.ds(base, BPW)], idx_v)
    pltpu.sync_copy(uniq_hbm, uniq_v)

    for i in range(BPW // L):
        y = idx_v[pl.ds(i * L, L)]
        pos = jnp.zeros((L,), jnp.int32)
        for s in _STEPS:
            cand = pos + s
            ok = cand <= C
            gidx = jnp.where(ok, cand, 1) - 1
            val = plsc.load_gather(uniq_v, [gidx])
            pos = jnp.where(ok & (val < y), cand, pos)
        idx_v[pl.ds(i * L, L)] = pos

    pltpu.sync_copy(idx_v, pred_hbm.at[pl.ds(base, BPW)])

    tbls = (anc_hbm, dec_hbm, mod_hbm)
    outs = (ml_hbm, dnc_hbm, mnc_hbm)
    HR = BPW // 2
    NT = 2 * NFULL

    def start_gather(t, s, col, h):
        pltpu.async_copy(
            tbls[t].at[:, pl.ds(col, CB)].at[idx_v.at[pl.ds(h * HR, HR)]],
            bufs.at[t * 4 + s], sems.at[t * 4 + s])

    def wait_gather(t, s):
        pltpu.make_async_copy(
            tbls[t].at[:, pl.ds(0, CB)].at[idx_v.at[pl.ds(0, HR)]],
            bufs.at[t * 4 + s], sems.at[t * 4 + s]).wait()

    def start_scatter(t, s, col, h):
        pltpu.async_copy(bufs.at[t * 4 + s],
                         outs[t].at[pl.ds(base + h * HR, HR), pl.ds(col, CB)],
                         sems.at[12 + t * 4 + s])

    def wait_scatter(t, s):
        pltpu.make_async_copy(
            bufs.at[t * 4 + s],
            outs[t].at[pl.ds(base, HR), pl.ds(0, CB)],
            sems.at[12 + t * 4 + s]).wait()

    for t in range(3):
        start_gather(t, 0, 0, 0)
        start_gather(t, 1, 0, 1)

    def quad_body(j, carry):
        for u in range(4):
            col = (2 * j + u // 2) * CB
            h = u % 2
            for t in range(3):
                wait_gather(t, u)
                start_scatter(t, u, col, h)
            for t in range(3):
                if u < 2:
                    @pl.when(j >= 1)
                    def _():
                        wait_scatter(t, u + 2)
                    start_gather(t, u + 2, col + CB, h)
                else:
                    wait_scatter(t, u - 2)

                    @pl.when(j < NT // 4 - 1)
                    def _():
                        start_gather(t, u - 2, col + CB, h)
        return carry

    lax.fori_loop(0, NT // 4, quad_body, 0)

    for t in range(3):
        wait_scatter(t, 2)
        wait_scatter(t, 3)


def _tc_tail_body(idx_ref, ta_ref, td_ref, tm_ref,
                  mli_ref, dnci_ref, mnci_ref,
                  mlo_ref, dnco_ref, mnco_ref):
    del mli_ref, dnci_ref, mnci_ref
    idx = idx_ref[0, 0, :]
    onehot = (lax.broadcasted_iota(jnp.int32, (BT, C), 1)
              == idx[:, None]).astype(jnp.float32)
    mlo_ref[...] = jnp.dot(onehot, ta_ref[...],
                           preferred_element_type=jnp.float32)
    dnco_ref[...] = jnp.dot(onehot, td_ref[...],
                            preferred_element_type=jnp.float32)
    mnco_ref[...] = jnp.dot(onehot, tm_ref[...],
                            preferred_element_type=jnp.float32)


BI = 1024
BJ = 512


def _tc_tr_body(mli_ref, dnci_ref, mnci_ref, mlo_ref, dnco_ref, mnco_ref):
    mlo_ref[...] = mli_ref[...].T
    dnco_ref[...] = dnci_ref[...].T
    mnco_ref[...] = mnci_ref[...].T


def kernel(y_n, unique_cell_types, ancestors, descendents, mod):
    out_type = (
        jax.ShapeDtypeStruct((B, C), jnp.float32),
        jax.ShapeDtypeStruct((B, C), jnp.float32),
        jax.ShapeDtypeStruct((B, C), jnp.float32),
        jax.ShapeDtypeStruct((B,), jnp.int32),
    )
    sc = pl.kernel(
        _sc_body,
        out_type=out_type,
        mesh=plsc.VectorSubcoreMesh(core_axis_name="c", subcore_axis_name="s"),
        compiler_params=pltpu.CompilerParams(needs_layout_passes=False,
                                             use_tc_tiling_on_sc=True),
        scratch_types=[
            pltpu.VMEM((C,), jnp.int32),
            pltpu.VMEM((BPW,), jnp.int32),
            pltpu.VMEM((12, BPW // 2, CB), jnp.float32),
            pltpu.SemaphoreType.DMA((24,)),
        ],
    )
    ml, dnc, mnc, pred = sc(y_n, unique_cell_types, ancestors, descendents,
                            mod)

    idx3 = pred.reshape(NBT, 1, BT)
    tail_spec = pl.BlockSpec((C, CB), lambda i: (0, NFULL))
    big_spec = pl.BlockSpec(memory_space=pl.ANY)
    out_spec = pl.BlockSpec((BT, CB), lambda i: (i, NFULL))
    tc = pl.pallas_call(
        _tc_tail_body,
        grid=(NBT,),
        in_specs=[
            pl.BlockSpec((1, 1, BT), lambda i: (i, 0, 0)),
            tail_spec, tail_spec, tail_spec,
            big_spec, big_spec, big_spec,
        ],
        out_specs=(out_spec, out_spec, out_spec),
        out_shape=(
            jax.ShapeDtypeStruct((B, C), jnp.float32),
            jax.ShapeDtypeStruct((B, C), jnp.float32),
            jax.ShapeDtypeStruct((B, C), jnp.float32),
        ),
        input_output_aliases={4: 0, 5: 1, 6: 2},
    )
    ml, dnc, mnc = tc(idx3, ancestors, descendents, mod, ml, dnc, mnc)

    in_spec = pl.BlockSpec((BI, BJ), lambda i, j: (i, j))
    tr_out_spec = pl.BlockSpec((BJ, BI), lambda i, j: (j, i))
    tr = pl.pallas_call(
        _tc_tr_body,
        grid=(B // BI, -(-C // BJ)),
        in_specs=[in_spec, in_spec, in_spec],
        out_specs=(tr_out_spec, tr_out_spec, tr_out_spec),
        out_shape=(
            jax.ShapeDtypeStruct((C, B), jnp.float32),
            jax.ShapeDtypeStruct((C, B), jnp.float32),
            jax.ShapeDtypeStruct((C, B), jnp.float32),
        ),
    )
    mlT, dncT, mncT = tr(ml, dnc, mnc)
    return (mlT.T, dncT.T, mncT.T, pred)


# speedup vs baseline: 1.1689x; 1.0217x over previous
import functools

import jax
import jax.numpy as jnp
from jax import lax
from jax.experimental import pallas as pl
from jax.experimental.pallas import tpu as pltpu, tpu_sc as plsc

C = 2600
B = 4096
L = 16
NC = 2
NS = 16
NW = NC * NS
BPW = B // NW
CB = 128
NFULL = C // CB
TAIL = C - NFULL * CB
BT = 512
NBT = B // BT

_STEPS = [2048, 1024, 512, 256, 128, 64, 32, 16, 8, 4, 2, 1]


def _sc_body(y_hbm, uniq_hbm, anc_hbm, dec_hbm, mod_hbm,
             ml_hbm, dnc_hbm, mnc_hbm, pred_hbm,
             uniq_v, idx_v, bufs, sems):
    wid = lax.axis_index("s") * NC + lax.axis_index("c")
    base = wid * BPW

    pltpu.sync_copy(y_hbm.at[pl.ds(base, BPW)], idx_v)
    pltpu.sync_copy(uniq_hbm, uniq_v)

    for i in range(BPW // L):
        y = idx_v[pl.ds(i * L, L)]
        pos = jnp.zeros((L,), jnp.int32)
        for s in _STEPS:
            cand = pos + s
            ok = cand <= C
            gidx = jnp.where(ok, cand, 1) - 1
            val = plsc.load_gather(uniq_v, [gidx])
            pos = jnp.where(ok & (val < y), cand, pos)
        idx_v[pl.ds(i * L, L)] = pos

    pltpu.sync_copy(idx_v, pred_hbm.at[pl.ds(base, BPW)])

    tbls = (anc_hbm, dec_hbm, mod_hbm)
    outs = (ml_hbm, dnc_hbm, mnc_hbm)
    HR = BPW // 2
    NT = 2 * NFULL

    def start_gather(t, s, col, h):
        pltpu.async_copy(
            tbls[t].at[:, pl.ds(col, CB)].at[idx_v.at[pl.ds(h * HR, HR)]],
            bufs.at[t * 4 + s], sems.at[t * 4 + s])

    def wait_gather(t, s):
        pltpu.make_async_copy(
            tbls[t].at[:, pl.ds(0, CB)].at[idx_v.at[pl.ds(0, HR)]],
            bufs.at[t * 4 + s], sems.at[t * 4 + s]).wait()

    def start_scatter(t, s, col, h):
        pltpu.async_copy(bufs.at[t * 4 + s],
                         outs[t].at[pl.ds(base + h * HR, HR), pl.ds(col, CB)],
                         sems.at[12 + t * 4 + s])

    def wait_scatter(t, s):
        pltpu.make_async_copy(
            bufs.at[t * 4 + s],
            outs[t].at[pl.ds(base, HR), pl.ds(0, CB)],
            sems.at[12 + t * 4 + s]).wait()

    for t in range(3):
        start_gather(t, 0, 0, 0)
        start_gather(t, 1, 0, 1)

    def quad_body(j, carry):
        for u in range(4):
            col = (2 * j + u // 2) * CB
            h = u % 2
            for t in range(3):
                wait_gather(t, u)
                start_scatter(t, u, col, h)
            for t in range(3):
                if u < 2:
                    @pl.when(j >= 1)
                    def _():
                        wait_scatter(t, u + 2)
                    start_gather(t, u + 2, col + CB, h)
                else:
                    wait_scatter(t, u - 2)

                    @pl.when(j < NT // 4 - 1)
                    def _():
                        start_gather(t, u - 2, col + CB, h)
        return carry

    lax.fori_loop(0, NT // 4, quad_body, 0)

    for t in range(3):
        wait_scatter(t, 2)
        wait_scatter(t, 3)


def _tc_tail_body(idx_ref, ta_ref, td_ref, tm_ref,
                  mli_ref, dnci_ref, mnci_ref,
                  mlo_ref, dnco_ref, mnco_ref):
    del mli_ref, dnci_ref, mnci_ref
    idx = idx_ref[0, 0, :]
    onehot = (lax.broadcasted_iota(jnp.int32, (BT, C), 1)
              == idx[:, None]).astype(jnp.float32)
    mlo_ref[...] = jnp.dot(onehot, ta_ref[...],
                           preferred_element_type=jnp.float32)
    dnco_ref[...] = jnp.dot(onehot, td_ref[...],
                            preferred_element_type=jnp.float32)
    mnco_ref[...] = jnp.dot(onehot, tm_ref[...],
                            preferred_element_type=jnp.float32)


BI = 1024
BJ = 896


def _tc_tr_body(mli_ref, dnci_ref, mnci_ref, mlo_ref, dnco_ref, mnco_ref):
    mlo_ref[...] = mli_ref[...].T
    dnco_ref[...] = dnci_ref[...].T
    mnco_ref[...] = mnci_ref[...].T


def kernel(y_n, unique_cell_types, ancestors, descendents, mod):
    out_type = (
        jax.ShapeDtypeStruct((B, C), jnp.float32),
        jax.ShapeDtypeStruct((B, C), jnp.float32),
        jax.ShapeDtypeStruct((B, C), jnp.float32),
        jax.ShapeDtypeStruct((B,), jnp.int32),
    )
    sc = pl.kernel(
        _sc_body,
        out_type=out_type,
        mesh=plsc.VectorSubcoreMesh(core_axis_name="c", subcore_axis_name="s"),
        compiler_params=pltpu.CompilerParams(needs_layout_passes=False,
                                             use_tc_tiling_on_sc=True),
        scratch_types=[
            pltpu.VMEM((C,), jnp.int32),
            pltpu.VMEM((BPW,), jnp.int32),
            pltpu.VMEM((12, BPW // 2, CB), jnp.float32),
            pltpu.SemaphoreType.DMA((24,)),
        ],
    )
    ml, dnc, mnc, pred = sc(y_n, unique_cell_types, ancestors, descendents,
                            mod)

    idx3 = pred.reshape(NBT, 1, BT)
    tail_spec = pl.BlockSpec((C, CB), lambda i: (0, NFULL))
    big_spec = pl.BlockSpec(memory_space=pl.ANY)
    out_spec = pl.BlockSpec((BT, CB), lambda i: (i, NFULL))
    tc = pl.pallas_call(
        _tc_tail_body,
        grid=(NBT,),
        in_specs=[
            pl.BlockSpec((1, 1, BT), lambda i: (i, 0, 0)),
            tail_spec, tail_spec, tail_spec,
            big_spec, big_spec, big_spec,
        ],
        out_specs=(out_spec, out_spec, out_spec),
        out_shape=(
            jax.ShapeDtypeStruct((B, C), jnp.float32),
            jax.ShapeDtypeStruct((B, C), jnp.float32),
            jax.ShapeDtypeStruct((B, C), jnp.float32),
        ),
        input_output_aliases={4: 0, 5: 1, 6: 2},
    )
    ml, dnc, mnc = tc(idx3, ancestors, descendents, mod, ml, dnc, mnc)

    in_spec = pl.BlockSpec((BI, BJ), lambda i, j: (i, j))
    tr_out_spec = pl.BlockSpec((BJ, BI), lambda i, j: (j, i))
    tr = pl.pallas_call(
        _tc_tr_body,
        grid=(B // BI, -(-C // BJ)),
        in_specs=[in_spec, in_spec, in_spec],
        out_specs=(tr_out_spec, tr_out_spec, tr_out_spec),
        out_shape=(
            jax.ShapeDtypeStruct((C, B), jnp.float32),
            jax.ShapeDtypeStruct((C, B), jnp.float32),
            jax.ShapeDtypeStruct((C, B), jnp.float32),
        ),
    )
    mlT, dncT, mncT = tr(ml, dnc, mnc)
    return (mlT.T, dncT.T, mncT.T, pred)
